# Initial kernel scaffold; baseline (speedup 1.0000x reference)
import jax
import jax.numpy as jnp
from jax.experimental import pallas as pl


def kernel(inputs, W_head, b_head, W0, b0, W1, b1, W2, b2, W_fuse, b_fuse):
    raise NotImplementedError("write your pallas kernel here")


# speedup vs baseline: 1.8147x; 1.8147x over previous
import functools

import jax
import jax.numpy as jnp
from jax import lax
from jax.experimental import pallas as pl
from jax.experimental.pallas import tpu as pltpu

_B, _N, _K = 4, 2048, 16
_C = 64
_CF = 1024
_F32 = jnp.float32
_BF16 = jnp.bfloat16
_HI = lax.Precision.HIGHEST


def _stage_kernel(cur_ref, rows_ref, w1_ref, w2_ref, b_ref, feat_ref, *,
                  d, residual, rows):
    xt = cur_ref[0]
    rws = rows_ref[0]
    xt_b = xt.astype(_BF16)
    rws_b = rws.astype(_BF16)
    w1_b = w1_ref[...].astype(_BF16)
    w2_b = w2_ref[...].astype(_BF16)

    inner = lax.dot_general(rws_b, xt_b, (((1,), (1,)), ((), ())),
                            preferred_element_type=_F32)
    sq_full = jnp.sum(xt * xt, axis=1)
    sq_rows = jnp.sum(rws * rws, axis=1)
    neg = 2.0 * inner - sq_rows[:, None] - sq_full[None, :]

    colid = lax.broadcasted_iota(jnp.int32, (rows, _N), 1)
    m = jnp.full((rows, _C), -jnp.inf, _F32)
    for r in range(15 * d + 1):
        mx = jnp.max(neg, axis=1, keepdims=True)
        cand = jnp.where(neg >= mx, colid, _N)
        amax = jnp.min(cand, axis=1, keepdims=True)
        hit = colid == amax
        if r % d == 0:
            oh = hit.astype(_F32)
            xj = lax.dot_general(oh, xt, (((1,), (0,)), ((), ())),
                                 preferred_element_type=_F32, precision=_HI)
            dif = (xj - rws).astype(_BF16)
            h = lax.dot_general(dif, w2_b, (((1,), (0,)), ((), ())),
                                preferred_element_type=_F32)
            m = jnp.maximum(m, h)
        neg = jnp.where(hit, -jnp.inf, neg)

    u = lax.dot_general(rws_b, w1_b, (((1,), (0,)), ((), ())),
                        preferred_element_type=_F32)
    out = jnp.maximum(u + m + b_ref[0], 0.0)
    if residual:
        out = out + rws
    feat_ref[0] = out


def _run_stage(cur, w1, w2, b, *, d, residual, rows=256):
    bsz, n, c = cur.shape
    grid = (bsz, n // rows)
    return pl.pallas_call(
        functools.partial(_stage_kernel, d=d, residual=residual, rows=rows),
        grid=grid,
        in_specs=[
            pl.BlockSpec((1, n, c), lambda b_, t: (b_, 0, 0)),
            pl.BlockSpec((1, rows, c), lambda b_, t: (b_, t, 0)),
            pl.BlockSpec((c, _C), lambda b_, t: (0, 0)),
            pl.BlockSpec((c, _C), lambda b_, t: (0, 0)),
            pl.BlockSpec((1, _C), lambda b_, t: (0, 0)),
        ],
        out_specs=pl.BlockSpec((1, rows, _C), lambda b_, t: (b_, t, 0)),
        out_shape=jax.ShapeDtypeStruct((bsz, n, _C), _F32),
        compiler_params=pltpu.CompilerParams(
            dimension_semantics=("parallel", "parallel")),
    )(cur, cur, w1, w2, b)


def _fusion_kernel(fcat_ref, wf_ref, bf_ref, pooled_ref):
    f = fcat_ref[0]
    z = lax.dot_general(f.astype(_BF16), wf_ref[...].astype(_BF16),
                        (((1,), (0,)), ((), ())),
                        preferred_element_type=_F32) + bf_ref[0]
    z = jnp.maximum(z, 0.0)
    pooled_ref[0, 0] = jnp.max(z, axis=0)


def _run_fusion(fcat, wf, bf):
    bsz, n, cc = fcat.shape
    return pl.pallas_call(
        _fusion_kernel,
        grid=(bsz,),
        in_specs=[
            pl.BlockSpec((1, n, cc), lambda b_: (b_, 0, 0)),
            pl.BlockSpec((cc, _CF), lambda b_: (0, 0)),
            pl.BlockSpec((1, _CF), lambda b_: (0, 0)),
        ],
        out_specs=pl.BlockSpec((1, 1, _CF), lambda b_: (b_, 0, 0)),
        out_shape=jax.ShapeDtypeStruct((bsz, 1, _CF), _F32),
        compiler_params=pltpu.CompilerParams(
            dimension_semantics=("parallel",)),
    )(fcat, wf, bf)


def kernel(inputs, W_head, b_head, W0, b0, W1, b1, W2, b2, W_fuse, b_fuse):
    x = inputs[..., 0]
    xt = jnp.transpose(x, (0, 2, 1))

    f0 = _run_stage(xt, W_head[:3], W_head[3:], b_head.reshape(1, -1),
                    d=1, residual=False)
    f1 = _run_stage(f0, W0[:_C], W0[_C:], b0.reshape(1, -1),
                    d=1, residual=True)
    f2 = _run_stage(f1, W1[:_C], W1[_C:], b1.reshape(1, -1),
                    d=2, residual=True)
    f3 = _run_stage(f2, W2[:_C], W2[_C:], b2.reshape(1, -1),
                    d=3, residual=True)

    fcat = jnp.concatenate([f0, f1, f2, f3], axis=-1)
    pooled = _run_fusion(fcat, W_fuse, b_fuse.reshape(1, -1))

    fcatT = jnp.transpose(fcat, (0, 2, 1))
    fus = jnp.broadcast_to(pooled[:, 0, :, None], (_B, _CF, _N))
    out = jnp.concatenate([fus, fcatT], axis=1)
    return out[..., None]


# speedup vs baseline: 6.0752x; 3.3477x over previous
import functools

import jax
import jax.numpy as jnp
from jax import lax
from jax.experimental import pallas as pl
from jax.experimental.pallas import tpu as pltpu
from jax.experimental.pallas import tpu_sc as plsc

_B, _N, _K = 4, 2048, 16
_C = 64
_CF = 1024
_F32 = jnp.float32
_BF16 = jnp.bfloat16
_HI = lax.Precision.HIGHEST


def _topk_kernel(cur_ref, rows_ref, idx_ref, *, d, rows):
    b = pl.program_id(0)
    xt = cur_ref[0]
    rws = rows_ref[0]
    xt_b = xt.astype(_BF16)
    rws_b = rws.astype(_BF16)

    inner = lax.dot_general(rws_b, xt_b, (((1,), (1,)), ((), ())),
                            preferred_element_type=_F32)
    sq_full = jnp.sum(xt * xt, axis=1)
    sq_rows = jnp.sum(rws * rws, axis=1)
    neg = 2.0 * inner - sq_rows[:, None] - sq_full[None, :]

    colid = lax.broadcasted_iota(jnp.int32, (rows, _N), 1)
    slotid = lax.broadcasted_iota(jnp.int32, (rows, _K), 1)
    acc = jnp.zeros((rows, _K), jnp.int32)
    for r in range(15 * d + 1):
        mx = jnp.max(neg, axis=1, keepdims=True)
        cand = jnp.where(neg >= mx, colid, _N)
        amax = jnp.min(cand, axis=1, keepdims=True)
        if r % d == 0:
            acc = jnp.where(slotid == (r // d), amax, acc)
        neg = jnp.where(colid == amax, -jnp.inf, neg)
    idx_ref[0] = acc + b * _N


def _run_topk(cur, *, d, rows=256):
    bsz, n, c = cur.shape
    grid = (bsz, n // rows)
    return pl.pallas_call(
        functools.partial(_topk_kernel, d=d, rows=rows),
        grid=grid,
        in_specs=[
            pl.BlockSpec((1, n, c), lambda b_, t: (b_, 0, 0)),
            pl.BlockSpec((1, rows, c), lambda b_, t: (b_, t, 0)),
        ],
        out_specs=pl.BlockSpec((1, rows, _K), lambda b_, t: (b_, t, 0)),
        out_shape=jax.ShapeDtypeStruct((bsz, n, _K), jnp.int32),
        compiler_params=pltpu.CompilerParams(
            dimension_semantics=("parallel", "parallel")),
    )(cur, cur)


_CHUNK = 128


def _make_gather(tot, ctab):
    mesh = plsc.VectorSubcoreMesh(core_axis_name="c", subcore_axis_name="s")
    info = plsc.get_sparse_core_info()
    nw = info.num_cores * info.num_subcores
    per_w = tot // nw
    n_iter = per_w // _CHUNK

    @functools.partial(
        pl.kernel, mesh=mesh,
        out_type=jax.ShapeDtypeStruct((tot, ctab), _F32),
        scratch_types=[
            pltpu.VMEM((_CHUNK,), jnp.int32),
            pltpu.VMEM((_CHUNK, ctab), _F32),
            pltpu.SemaphoreType.DMA,
        ],
    )
    def gather(table_hbm, idx_hbm, out_hbm, idx_v, rows_v, sem):
        wid = lax.axis_index("s") * info.num_cores + lax.axis_index("c")
        base = wid * per_w

        def body(i, carry):
            off = base + i * _CHUNK
            pltpu.sync_copy(idx_hbm.at[pl.ds(off, _CHUNK)], idx_v)
            pltpu.async_copy(table_hbm.at[idx_v], rows_v, sem).wait()
            pltpu.sync_copy(rows_v, out_hbm.at[pl.ds(off, _CHUNK)])
            return carry

        lax.fori_loop(0, n_iter, body, 0)

    return gather


def _edge_kernel(xj_ref, rows_ref, w1_ref, w2_ref, b_ref, res_ref, feat_ref,
                 *, rows, residual):
    xj = xj_ref[0]
    rws = rows_ref[0]
    w1_b = w1_ref[...].astype(_BF16)
    w2_b = w2_ref[...].astype(_BF16)
    xj3 = xj.reshape(rows, _K, xj.shape[-1])

    m = jnp.full((rows, _C), -jnp.inf, _F32)
    for k in range(_K):
        dif = (xj3[:, k, :] - rws).astype(_BF16)
        h = lax.dot_general(dif, w2_b, (((1,), (0,)), ((), ())),
                            preferred_element_type=_F32)
        m = jnp.maximum(m, h)

    u = lax.dot_general(rws.astype(_BF16), w1_b, (((1,), (0,)), ((), ())),
                        preferred_element_type=_F32)
    out = jnp.maximum(u + m + b_ref[0], 0.0)
    if residual:
        out = out + res_ref[0]
    feat_ref[0] = out


def _run_edge(xj, cur_pad, res, w1, w2, b, *, residual, rows=256):
    bsz, n, ctab = cur_pad.shape
    grid = (bsz, n // rows)
    return pl.pallas_call(
        functools.partial(_edge_kernel, rows=rows, residual=residual),
        grid=grid,
        in_specs=[
            pl.BlockSpec((1, rows * _K, ctab), lambda b_, t: (b_, t, 0)),
            pl.BlockSpec((1, rows, ctab), lambda b_, t: (b_, t, 0)),
            pl.BlockSpec((ctab, _C), lambda b_, t: (0, 0)),
            pl.BlockSpec((ctab, _C), lambda b_, t: (0, 0)),
            pl.BlockSpec((1, _C), lambda b_, t: (0, 0)),
            pl.BlockSpec((1, rows, _C), lambda b_, t: (b_, t, 0)),
        ],
        out_specs=pl.BlockSpec((1, rows, _C), lambda b_, t: (b_, t, 0)),
        out_shape=jax.ShapeDtypeStruct((bsz, n, _C), _F32),
        compiler_params=pltpu.CompilerParams(
            dimension_semantics=("parallel", "parallel")),
    )(xj, cur_pad, w1, w2, b, res)


def _stage(cur_pad, res, w1, w2, b, *, d, residual):
    bsz, n, ctab = cur_pad.shape
    idx = _run_topk(cur_pad, d=d)
    table = cur_pad.reshape(bsz * n, ctab)
    gath = _make_gather(bsz * n * _K, ctab)
    xj = gath(table, idx.reshape(bsz * n * _K))
    xj = xj.reshape(bsz, n * _K, ctab)
    return _run_edge(xj, cur_pad, res, w1, w2, b, residual=residual)


def _fusion_kernel(fcat_ref, wf_ref, bf_ref, pooled_ref):
    f = fcat_ref[0]
    z = lax.dot_general(f.astype(_BF16), wf_ref[...].astype(_BF16),
                        (((1,), (0,)), ((), ())),
                        preferred_element_type=_F32) + bf_ref[0]
    z = jnp.maximum(z, 0.0)
    pooled_ref[0, 0] = jnp.max(z, axis=0)


def _run_fusion(fcat, wf, bf):
    bsz, n, cc = fcat.shape
    return pl.pallas_call(
        _fusion_kernel,
        grid=(bsz,),
        in_specs=[
            pl.BlockSpec((1, n, cc), lambda b_: (b_, 0, 0)),
            pl.BlockSpec((cc, _CF), lambda b_: (0, 0)),
            pl.BlockSpec((1, _CF), lambda b_: (0, 0)),
        ],
        out_specs=pl.BlockSpec((1, 1, _CF), lambda b_: (b_, 0, 0)),
        out_shape=jax.ShapeDtypeStruct((bsz, 1, _CF), _F32),
        compiler_params=pltpu.CompilerParams(
            dimension_semantics=("parallel",)),
    )(fcat, wf, bf)


def kernel(inputs, W_head, b_head, W0, b0, W1, b1, W2, b2, W_fuse, b_fuse):
    x = inputs[..., 0]
    xt = jnp.transpose(x, (0, 2, 1))
    pad = lambda a: jnp.pad(a, ((0, 0), (0, 0), (0, 128 - a.shape[-1])))
    padw = lambda w: jnp.pad(w, ((0, 128 - w.shape[0]), (0, 0)))
    xt_pad = pad(xt)
    w1h, w2h = padw(W_head[:3]), padw(W_head[3:])

    zeros = jnp.zeros((_B, _N, _C), _F32)
    f0 = _stage(xt_pad, zeros, w1h, w2h, b_head.reshape(1, -1),
                d=1, residual=False)
    f1 = _stage(pad(f0), f0, padw(W0[:_C]), padw(W0[_C:]), b0.reshape(1, -1),
                d=1, residual=True)
    f2 = _stage(pad(f1), f1, padw(W1[:_C]), padw(W1[_C:]), b1.reshape(1, -1),
                d=2, residual=True)
    f3 = _stage(pad(f2), f2, padw(W2[:_C]), padw(W2[_C:]), b2.reshape(1, -1),
                d=3, residual=True)

    fcat = jnp.concatenate([f0, f1, f2, f3], axis=-1)
    pooled = _run_fusion(fcat, W_fuse, b_fuse.reshape(1, -1))

    fcatT = jnp.transpose(fcat, (0, 2, 1))
    fus = jnp.broadcast_to(pooled[:, 0, :, None], (_B, _CF, _N))
    out = jnp.concatenate([fus, fcatT], axis=1)
    return out[..., None]


# speedup vs baseline: 7.8048x; 1.2847x over previous
import functools

import jax
import jax.numpy as jnp
from jax import lax
from jax.experimental import pallas as pl
from jax.experimental.pallas import tpu as pltpu
from jax.experimental.pallas import tpu_sc as plsc

_B, _N, _K = 4, 2048, 16
_C = 64
_CF = 1024
_F32 = jnp.float32
_BF16 = jnp.bfloat16
_HI = lax.Precision.HIGHEST


def _topk_kernel(cur_ref, rows_ref, idx_ref, *, d, rows):
    b = pl.program_id(0)
    xt = cur_ref[0]
    rws = rows_ref[0]
    xt_b = xt.astype(_BF16)
    rws_b = rws.astype(_BF16)

    inner = lax.dot_general(rws_b, xt_b, (((1,), (1,)), ((), ())),
                            preferred_element_type=_F32)
    sq_full = jnp.sum(xt * xt, axis=1)
    sq_rows = jnp.sum(rws * rws, axis=1)
    neg = 2.0 * inner - sq_rows[:, None] - sq_full[None, :]

    colid = lax.broadcasted_iota(jnp.int32, (rows, _N), 1)
    slotid = lax.broadcasted_iota(jnp.int32, (rows, _K), 1)
    acc = jnp.zeros((rows, _K), jnp.int32)
    vcur = jnp.full((rows, 1), jnp.inf, _F32)
    for r in range(15 * d + 1):
        vcur = jnp.max(jnp.where(neg < vcur, neg, -jnp.inf), axis=1,
                       keepdims=True)
        if r % d == 0:
            amax = jnp.min(jnp.where(neg == vcur, colid, _N - 1), axis=1,
                           keepdims=True)
            acc = jnp.where(slotid == (r // d), amax, acc)
    idx_ref[0] = acc + b * _N


def _run_topk(cur, *, d, rows=256):
    bsz, n, c = cur.shape
    grid = (bsz, n // rows)
    return pl.pallas_call(
        functools.partial(_topk_kernel, d=d, rows=rows),
        grid=grid,
        in_specs=[
            pl.BlockSpec((1, n, c), lambda b_, t: (b_, 0, 0)),
            pl.BlockSpec((1, rows, c), lambda b_, t: (b_, t, 0)),
        ],
        out_specs=pl.BlockSpec((1, rows, _K), lambda b_, t: (b_, t, 0)),
        out_shape=jax.ShapeDtypeStruct((bsz, n, _K), jnp.int32),
        compiler_params=pltpu.CompilerParams(
            dimension_semantics=("parallel", "parallel")),
    )(cur, cur)


_CHUNK = 128


def _make_gather(tot, ctab):
    mesh = plsc.VectorSubcoreMesh(core_axis_name="c", subcore_axis_name="s")
    info = plsc.get_sparse_core_info()
    nw = info.num_cores * info.num_subcores
    per_w = tot // nw
    n_iter = per_w // _CHUNK

    @functools.partial(
        pl.kernel, mesh=mesh,
        out_type=jax.ShapeDtypeStruct((tot, ctab), _F32),
        scratch_types=[
            pltpu.VMEM((_CHUNK,), jnp.int32),
            pltpu.VMEM((_CHUNK, ctab), _F32),
            pltpu.SemaphoreType.DMA,
        ],
    )
    def gather(table_hbm, idx_hbm, out_hbm, idx_v, rows_v, sem):
        wid = lax.axis_index("s") * info.num_cores + lax.axis_index("c")
        base = wid * per_w

        def body(i, carry):
            off = base + i * _CHUNK
            pltpu.sync_copy(idx_hbm.at[pl.ds(off, _CHUNK)], idx_v)
            pltpu.async_copy(table_hbm.at[idx_v], rows_v, sem).wait()
            pltpu.sync_copy(rows_v, out_hbm.at[pl.ds(off, _CHUNK)])
            return carry

        lax.fori_loop(0, n_iter, body, 0)

    return gather


def _edge_kernel(xj_ref, rows_ref, w1_ref, w2_ref, b_ref, res_ref, feat_ref,
                 *, rows, residual):
    xj = xj_ref[0]
    rws = rows_ref[0]
    w1_b = w1_ref[...].astype(_BF16)
    w2_b = w2_ref[...].astype(_BF16)
    xj3 = xj.reshape(rows, _K, xj.shape[-1])

    m = jnp.full((rows, _C), -jnp.inf, _F32)
    for k in range(_K):
        dif = (xj3[:, k, :] - rws).astype(_BF16)
        h = lax.dot_general(dif, w2_b, (((1,), (0,)), ((), ())),
                            preferred_element_type=_F32)
        m = jnp.maximum(m, h)

    u = lax.dot_general(rws.astype(_BF16), w1_b, (((1,), (0,)), ((), ())),
                        preferred_element_type=_F32)
    out = jnp.maximum(u + m + b_ref[0], 0.0)
    if residual:
        out = out + res_ref[0]
    feat_ref[0] = out


def _run_edge(xj, cur_pad, res, w1, w2, b, *, residual, rows=256):
    bsz, n, ctab = cur_pad.shape
    grid = (bsz, n // rows)
    return pl.pallas_call(
        functools.partial(_edge_kernel, rows=rows, residual=residual),
        grid=grid,
        in_specs=[
            pl.BlockSpec((1, rows * _K, ctab), lambda b_, t: (b_, t, 0)),
            pl.BlockSpec((1, rows, ctab), lambda b_, t: (b_, t, 0)),
            pl.BlockSpec((ctab, _C), lambda b_, t: (0, 0)),
            pl.BlockSpec((ctab, _C), lambda b_, t: (0, 0)),
            pl.BlockSpec((1, _C), lambda b_, t: (0, 0)),
            pl.BlockSpec((1, rows, _C), lambda b_, t: (b_, t, 0)),
        ],
        out_specs=pl.BlockSpec((1, rows, _C), lambda b_, t: (b_, t, 0)),
        out_shape=jax.ShapeDtypeStruct((bsz, n, _C), _F32),
        compiler_params=pltpu.CompilerParams(
            dimension_semantics=("parallel", "parallel")),
    )(xj, cur_pad, w1, w2, b, res)


def _stage(cur_pad, knn_in, res, w1, w2, b, *, d, residual):
    bsz, n, ctab = cur_pad.shape
    idx = _run_topk(knn_in, d=d)
    table = cur_pad.reshape(bsz * n, ctab)
    gath = _make_gather(bsz * n * _K, ctab)
    xj = gath(table, idx.reshape(bsz * n * _K))
    xj = xj.reshape(bsz, n * _K, ctab)
    return _run_edge(xj, cur_pad, res, w1, w2, b, residual=residual)


def _fusion_kernel(fcat_ref, wf_ref, bf_ref, pooled_ref):
    f = fcat_ref[0]
    z = lax.dot_general(f.astype(_BF16), wf_ref[...].astype(_BF16),
                        (((1,), (0,)), ((), ())),
                        preferred_element_type=_F32) + bf_ref[0]
    z = jnp.maximum(z, 0.0)
    pooled_ref[0, 0] = jnp.max(z, axis=0)


def _run_fusion(fcat, wf, bf):
    bsz, n, cc = fcat.shape
    return pl.pallas_call(
        _fusion_kernel,
        grid=(bsz,),
        in_specs=[
            pl.BlockSpec((1, n, cc), lambda b_: (b_, 0, 0)),
            pl.BlockSpec((cc, _CF), lambda b_: (0, 0)),
            pl.BlockSpec((1, _CF), lambda b_: (0, 0)),
        ],
        out_specs=pl.BlockSpec((1, 1, _CF), lambda b_: (b_, 0, 0)),
        out_shape=jax.ShapeDtypeStruct((bsz, 1, _CF), _F32),
        compiler_params=pltpu.CompilerParams(
            dimension_semantics=("parallel",)),
    )(fcat, wf, bf)


def kernel(inputs, W_head, b_head, W0, b0, W1, b1, W2, b2, W_fuse, b_fuse):
    x = inputs[..., 0]
    xt = jnp.transpose(x, (0, 2, 1))
    pad = lambda a: jnp.pad(a, ((0, 0), (0, 0), (0, 128 - a.shape[-1])))
    padw = lambda w: jnp.pad(w, ((0, 128 - w.shape[0]), (0, 0)))
    xt_pad = pad(xt)
    w1h, w2h = padw(W_head[:3]), padw(W_head[3:])

    zeros = jnp.zeros((_B, _N, _C), _F32)
    f0 = _stage(xt_pad, xt, zeros, w1h, w2h, b_head.reshape(1, -1),
                d=1, residual=False)
    f1 = _stage(pad(f0), f0, f0, padw(W0[:_C]), padw(W0[_C:]),
                b0.reshape(1, -1), d=1, residual=True)
    f2 = _stage(pad(f1), f1, f1, padw(W1[:_C]), padw(W1[_C:]),
                b1.reshape(1, -1), d=2, residual=True)
    f3 = _stage(pad(f2), f2, f2, padw(W2[:_C]), padw(W2[_C:]),
                b2.reshape(1, -1), d=3, residual=True)

    fcat = jnp.concatenate([f0, f1, f2, f3], axis=-1)
    pooled = _run_fusion(fcat, W_fuse, b_fuse.reshape(1, -1))

    fcatT = jnp.transpose(fcat, (0, 2, 1))
    fus = jnp.broadcast_to(pooled[:, 0, :, None], (_B, _CF, _N))
    out = jnp.concatenate([fus, fcatT], axis=1)
    return out[..., None]


# speedup vs baseline: 8.2616x; 1.0585x over previous
import functools

import jax
import jax.numpy as jnp
from jax import lax
from jax.experimental import pallas as pl
from jax.experimental.pallas import tpu as pltpu
from jax.experimental.pallas import tpu_sc as plsc

_B, _N, _K = 4, 2048, 16
_C = 64
_CF = 1024
_F32 = jnp.float32
_BF16 = jnp.bfloat16
_HI = lax.Precision.HIGHEST


def _topk_kernel(cur_ref, rows_ref, idx_ref, *, d, rows):
    b = pl.program_id(0)
    xt = cur_ref[0]
    rws = rows_ref[0]
    xt_b = xt.astype(_BF16)
    rws_b = rws.astype(_BF16)

    inner = lax.dot_general(rws_b, xt_b, (((1,), (1,)), ((), ())),
                            preferred_element_type=_F32)
    sq_full = jnp.sum(xt * xt, axis=1)
    sq_rows = jnp.sum(rws * rws, axis=1)
    neg = 2.0 * inner - sq_rows[:, None] - sq_full[None, :]

    colid = lax.broadcasted_iota(jnp.int32, (rows, _N), 1)
    slotid = lax.broadcasted_iota(jnp.int32, (rows, _K), 1)
    acc = jnp.zeros((rows, _K), jnp.int32)
    vcur = jnp.full((rows, 1), jnp.inf, _F32)
    for r in range(15 * d + 1):
        vcur = jnp.max(jnp.where(neg < vcur, neg, -jnp.inf), axis=1,
                       keepdims=True)
        if r % d == 0:
            amax = jnp.min(jnp.where(neg == vcur, colid, _N - 1), axis=1,
                           keepdims=True)
            acc = jnp.where(slotid == (r // d), amax, acc)
    idx_ref[0] = acc + b * _N


def _run_topk(cur, *, d, rows=256):
    bsz, n, c = cur.shape
    grid = (bsz, n // rows)
    return pl.pallas_call(
        functools.partial(_topk_kernel, d=d, rows=rows),
        grid=grid,
        in_specs=[
            pl.BlockSpec((1, n, c), lambda b_, t: (b_, 0, 0)),
            pl.BlockSpec((1, rows, c), lambda b_, t: (b_, t, 0)),
        ],
        out_specs=pl.BlockSpec((1, rows, _K), lambda b_, t: (b_, t, 0)),
        out_shape=jax.ShapeDtypeStruct((bsz, n, _K), jnp.int32),
        compiler_params=pltpu.CompilerParams(
            dimension_semantics=("parallel", "parallel")),
    )(cur, cur)


_CHUNK = 128


def _make_gather(tot, ctab):
    mesh = plsc.VectorSubcoreMesh(core_axis_name="c", subcore_axis_name="s")
    info = plsc.get_sparse_core_info()
    nw = info.num_cores * info.num_subcores
    per_w = tot // nw
    n_iter = per_w // _CHUNK

    @functools.partial(
        pl.kernel, mesh=mesh,
        out_type=jax.ShapeDtypeStruct((tot, ctab), _F32),
        scratch_types=[
            pltpu.VMEM((_CHUNK,), jnp.int32),
            pltpu.VMEM((_CHUNK,), jnp.int32),
            pltpu.VMEM((_CHUNK, ctab), _F32),
            pltpu.VMEM((_CHUNK, ctab), _F32),
            pltpu.SemaphoreType.DMA,
            pltpu.SemaphoreType.DMA,
            pltpu.SemaphoreType.DMA,
            pltpu.SemaphoreType.DMA,
        ],
    )
    def gather(table_hbm, idx_hbm, out_hbm, i0, i1, r0, r1,
               sg0, sg1, sw0, sw1):
        wid = lax.axis_index("s") * info.num_cores + lax.axis_index("c")
        base = wid * per_w
        idx_v = (i0, i1)
        rows_v = (r0, r1)
        sg = (sg0, sg1)
        sw = (sw0, sw1)
        gh = [None, None]
        wh = [None, None]
        for i in range(n_iter):
            bf = i & 1
            if wh[bf] is not None:
                wh[bf].wait()
            off = base + i * _CHUNK
            pltpu.sync_copy(idx_hbm.at[pl.ds(off, _CHUNK)], idx_v[bf])
            gh[bf] = pltpu.async_copy(table_hbm.at[idx_v[bf]], rows_v[bf],
                                      sg[bf])
            pb = bf ^ 1
            if gh[pb] is not None:
                gh[pb].wait()
                po = base + (i - 1) * _CHUNK
                wh[pb] = pltpu.async_copy(
                    rows_v[pb], out_hbm.at[pl.ds(po, _CHUNK)], sw[pb])
        lb = (n_iter - 1) & 1
        gh[lb].wait()
        lo = base + (n_iter - 1) * _CHUNK
        wh[lb] = pltpu.async_copy(rows_v[lb], out_hbm.at[pl.ds(lo, _CHUNK)],
                                  sw[lb])
        wh[lb ^ 1].wait()
        wh[lb].wait()

    return gather


def _edge_kernel(xj_ref, rows_ref, w1_ref, w2_ref, b_ref, res_ref, feat_ref,
                 *, rows, residual):
    xj = xj_ref[0][:, :_C]
    rws = rows_ref[0][:, :_C]
    w1_b = w1_ref[...].astype(_BF16)
    w2_b = w2_ref[...].astype(_BF16)
    xj3 = xj.reshape(rows, _K, _C)

    m = jnp.full((rows, _C), -jnp.inf, _F32)
    for k in range(_K):
        dif = (xj3[:, k, :] - rws).astype(_BF16)
        h = lax.dot_general(dif, w2_b, (((1,), (0,)), ((), ())),
                            preferred_element_type=_F32)
        m = jnp.maximum(m, h)

    u = lax.dot_general(rws.astype(_BF16), w1_b, (((1,), (0,)), ((), ())),
                        preferred_element_type=_F32)
    out = jnp.maximum(u + m + b_ref[0], 0.0)
    if residual:
        out = out + res_ref[0]
    feat_ref[0] = out


def _run_edge(xj, cur_pad, res, w1, w2, b, *, residual, rows=256):
    bsz, n, ctab = cur_pad.shape
    grid = (bsz, n // rows)
    return pl.pallas_call(
        functools.partial(_edge_kernel, rows=rows, residual=residual),
        grid=grid,
        in_specs=[
            pl.BlockSpec((1, rows * _K, ctab), lambda b_, t: (b_, t, 0)),
            pl.BlockSpec((1, rows, ctab), lambda b_, t: (b_, t, 0)),
            pl.BlockSpec((_C, _C), lambda b_, t: (0, 0)),
            pl.BlockSpec((_C, _C), lambda b_, t: (0, 0)),
            pl.BlockSpec((1, _C), lambda b_, t: (0, 0)),
            pl.BlockSpec((1, rows, _C), lambda b_, t: (b_, t, 0)),
        ],
        out_specs=pl.BlockSpec((1, rows, _C), lambda b_, t: (b_, t, 0)),
        out_shape=jax.ShapeDtypeStruct((bsz, n, _C), _F32),
        compiler_params=pltpu.CompilerParams(
            dimension_semantics=("parallel", "parallel")),
    )(xj, cur_pad, w1, w2, b, res)


def _stage(cur_pad, knn_in, res, w1, w2, b, *, d, residual):
    bsz, n, ctab = cur_pad.shape
    idx = _run_topk(knn_in, d=d)
    table = cur_pad.reshape(bsz * n, ctab)
    gath = _make_gather(bsz * n * _K, ctab)
    xj = gath(table, idx.reshape(bsz * n * _K))
    xj = xj.reshape(bsz, n * _K, ctab)
    return _run_edge(xj, cur_pad, res, w1, w2, b, residual=residual)


def _fusion_kernel(fcat_ref, wf_ref, bf_ref, pooled_ref):
    f = fcat_ref[0]
    z = lax.dot_general(f.astype(_BF16), wf_ref[...].astype(_BF16),
                        (((1,), (0,)), ((), ())),
                        preferred_element_type=_F32) + bf_ref[0]
    z = jnp.maximum(z, 0.0)
    pooled_ref[0, 0] = jnp.max(z, axis=0)


def _run_fusion(fcat, wf, bf):
    bsz, n, cc = fcat.shape
    return pl.pallas_call(
        _fusion_kernel,
        grid=(bsz,),
        in_specs=[
            pl.BlockSpec((1, n, cc), lambda b_: (b_, 0, 0)),
            pl.BlockSpec((cc, _CF), lambda b_: (0, 0)),
            pl.BlockSpec((1, _CF), lambda b_: (0, 0)),
        ],
        out_specs=pl.BlockSpec((1, 1, _CF), lambda b_: (b_, 0, 0)),
        out_shape=jax.ShapeDtypeStruct((bsz, 1, _CF), _F32),
        compiler_params=pltpu.CompilerParams(
            dimension_semantics=("parallel",)),
    )(fcat, wf, bf)


def kernel(inputs, W_head, b_head, W0, b0, W1, b1, W2, b2, W_fuse, b_fuse):
    x = inputs[..., 0]
    xt = jnp.transpose(x, (0, 2, 1))
    pad = lambda a: jnp.pad(a, ((0, 0), (0, 0), (0, 128 - a.shape[-1])))
    padw = lambda w: jnp.pad(w, ((0, _C - w.shape[0]), (0, 0)))
    xt_pad = pad(xt)
    w1h, w2h = padw(W_head[:3]), padw(W_head[3:])

    zeros = jnp.zeros((_B, _N, _C), _F32)
    f0 = _stage(xt_pad, xt, zeros, w1h, w2h, b_head.reshape(1, -1),
                d=1, residual=False)
    f1 = _stage(pad(f0), f0, f0, W0[:_C], W0[_C:],
                b0.reshape(1, -1), d=1, residual=True)
    f2 = _stage(pad(f1), f1, f1, W1[:_C], W1[_C:],
                b1.reshape(1, -1), d=2, residual=True)
    f3 = _stage(pad(f2), f2, f2, W2[:_C], W2[_C:],
                b2.reshape(1, -1), d=3, residual=True)

    fcat = jnp.concatenate([f0, f1, f2, f3], axis=-1)
    pooled = _run_fusion(fcat, W_fuse, b_fuse.reshape(1, -1))

    fcatT = jnp.transpose(fcat, (0, 2, 1))
    fus = jnp.broadcast_to(pooled[:, 0, :, None], (_B, _CF, _N))
    out = jnp.concatenate([fus, fcatT], axis=1)
    return out[..., None]


# speedup vs baseline: 8.7440x; 1.0584x over previous
import functools

import jax
import jax.numpy as jnp
from jax import lax
from jax.experimental import pallas as pl
from jax.experimental.pallas import tpu as pltpu
from jax.experimental.pallas import tpu_sc as plsc

_B, _N, _K = 4, 2048, 16
_C = 64
_CF = 1024
_F32 = jnp.float32
_BF16 = jnp.bfloat16
_HI = lax.Precision.HIGHEST


def _topk_kernel(cur_ref, rows_ref, idx_ref, *, d, rows, b0):
    xt = cur_ref[0]
    rws = rows_ref[0]
    xt_b = xt.astype(_BF16)
    rws_b = rws.astype(_BF16)

    inner = lax.dot_general(rws_b, xt_b, (((1,), (1,)), ((), ())),
                            preferred_element_type=_F32)
    sq_full = jnp.sum(xt * xt, axis=1)
    sq_rows = jnp.sum(rws * rws, axis=1)
    neg = 2.0 * inner - sq_rows[:, None] - sq_full[None, :]

    colid = lax.broadcasted_iota(jnp.int32, (rows, _N), 1)
    slotid = lax.broadcasted_iota(jnp.int32, (rows, _K), 1)
    acc = jnp.zeros((rows, _K), jnp.int32)
    vcur = jnp.full((rows, 1), jnp.inf, _F32)
    for r in range(15 * d + 1):
        vcur = jnp.max(jnp.where(neg < vcur, neg, -jnp.inf), axis=1,
                       keepdims=True)
        if r % d == 0:
            amax = jnp.min(jnp.where(neg == vcur, colid, _N - 1), axis=1,
                           keepdims=True)
            acc = jnp.where(slotid == (r // d), amax, acc)
    idx_ref[0] = acc + b0 * _N


def _run_topk(cur, b0, *, d, rows=256):
    bsz, n, c = cur.shape
    grid = (n // rows,)
    return pl.pallas_call(
        functools.partial(_topk_kernel, d=d, rows=rows, b0=b0),
        grid=grid,
        in_specs=[
            pl.BlockSpec((1, n, c), lambda t: (b0, 0, 0)),
            pl.BlockSpec((1, rows, c), lambda t: (b0, t, 0)),
        ],
        out_specs=pl.BlockSpec((1, rows, _K), lambda t: (0, t, 0)),
        out_shape=jax.ShapeDtypeStruct((1, n, _K), jnp.int32),
        compiler_params=pltpu.CompilerParams(
            dimension_semantics=("arbitrary",)),
    )(cur, cur)


_CHUNK = 128


def _make_gather(tot, ctab):
    mesh = plsc.VectorSubcoreMesh(core_axis_name="c", subcore_axis_name="s")
    info = plsc.get_sparse_core_info()
    nw = info.num_cores * info.num_subcores
    per_w = tot // nw
    n_iter = per_w // _CHUNK

    @functools.partial(
        pl.kernel, mesh=mesh,
        out_type=jax.ShapeDtypeStruct((tot, ctab), _F32),
        scratch_types=[
            pltpu.VMEM((_CHUNK,), jnp.int32),
            pltpu.VMEM((_CHUNK,), jnp.int32),
            pltpu.VMEM((_CHUNK, ctab), _F32),
            pltpu.VMEM((_CHUNK, ctab), _F32),
            pltpu.SemaphoreType.DMA,
            pltpu.SemaphoreType.DMA,
            pltpu.SemaphoreType.DMA,
            pltpu.SemaphoreType.DMA,
        ],
    )
    def gather(table_hbm, idx_hbm, out_hbm, i0, i1, r0, r1,
               sg0, sg1, sw0, sw1):
        wid = lax.axis_index("s") * info.num_cores + lax.axis_index("c")
        base = wid * per_w
        idx_v = (i0, i1)
        rows_v = (r0, r1)
        sg = (sg0, sg1)
        sw = (sw0, sw1)
        gh = [None, None]
        wh = [None, None]
        for i in range(n_iter):
            bf = i & 1
            if wh[bf] is not None:
                wh[bf].wait()
            off = base + i * _CHUNK
            pltpu.sync_copy(idx_hbm.at[pl.ds(off, _CHUNK)], idx_v[bf])
            gh[bf] = pltpu.async_copy(table_hbm.at[idx_v[bf]], rows_v[bf],
                                      sg[bf])
            pb = bf ^ 1
            if gh[pb] is not None:
                gh[pb].wait()
                po = base + (i - 1) * _CHUNK
                wh[pb] = pltpu.async_copy(
                    rows_v[pb], out_hbm.at[pl.ds(po, _CHUNK)], sw[pb])
        lb = (n_iter - 1) & 1
        gh[lb].wait()
        lo = base + (n_iter - 1) * _CHUNK
        wh[lb] = pltpu.async_copy(rows_v[lb], out_hbm.at[pl.ds(lo, _CHUNK)],
                                  sw[lb])
        wh[lb ^ 1].wait()
        wh[lb].wait()

    return gather


def _edge_kernel(xj_ref, rows_ref, w1_ref, w2_ref, b_ref, res_ref, feat_ref,
                 *, rows, residual):
    xj = xj_ref[:, :_C]
    rws = rows_ref[0][:, :_C]
    w1_b = w1_ref[...].astype(_BF16)
    w2_b = w2_ref[...].astype(_BF16)
    xj3 = xj.reshape(rows, _K, _C)

    m = jnp.full((rows, _C), -jnp.inf, _F32)
    for k in range(_K):
        dif = (xj3[:, k, :] - rws).astype(_BF16)
        h = lax.dot_general(dif, w2_b, (((1,), (0,)), ((), ())),
                            preferred_element_type=_F32)
        m = jnp.maximum(m, h)

    u = lax.dot_general(rws.astype(_BF16), w1_b, (((1,), (0,)), ((), ())),
                        preferred_element_type=_F32)
    out = jnp.maximum(u + m + b_ref[0], 0.0)
    if residual:
        out = out + res_ref[0]
    feat_ref[0] = out


def _run_edge(xj, cur_pad, res, w1, w2, b, b0, *, residual, rows=256):
    bsz, n, ctab = cur_pad.shape
    grid = (n // rows,)
    return pl.pallas_call(
        functools.partial(_edge_kernel, rows=rows, residual=residual),
        grid=grid,
        in_specs=[
            pl.BlockSpec((rows * _K, ctab), lambda t: (t, 0)),
            pl.BlockSpec((1, rows, ctab), lambda t: (b0, t, 0)),
            pl.BlockSpec((_C, _C), lambda t: (0, 0)),
            pl.BlockSpec((_C, _C), lambda t: (0, 0)),
            pl.BlockSpec((1, _C), lambda t: (0, 0)),
            pl.BlockSpec((1, rows, _C), lambda t: (b0, t, 0)),
        ],
        out_specs=pl.BlockSpec((1, rows, _C), lambda t: (0, t, 0)),
        out_shape=jax.ShapeDtypeStruct((1, n, _C), _F32),
        compiler_params=pltpu.CompilerParams(
            dimension_semantics=("arbitrary",)),
    )(xj, cur_pad, w1, w2, b, res)


def _stage(cur_pad, knn_in, res, w1, w2, b, *, d, residual):
    bsz, n, ctab = cur_pad.shape
    table = cur_pad.reshape(bsz * n, ctab)
    gath = _make_gather(n * _K, ctab)
    feats = []
    for b0 in range(bsz):
        idx_b = _run_topk(knn_in, b0, d=d)
        xj_b = gath(table, idx_b.reshape(n * _K))
        feats.append(_run_edge(xj_b, cur_pad, res, w1, w2, b, b0,
                               residual=residual))
    return jnp.concatenate(feats, axis=0)


def _fusion_kernel(fcat_ref, wf_ref, bf_ref, pooled_ref):
    f = fcat_ref[0]
    z = lax.dot_general(f.astype(_BF16), wf_ref[...].astype(_BF16),
                        (((1,), (0,)), ((), ())),
                        preferred_element_type=_F32) + bf_ref[0]
    z = jnp.maximum(z, 0.0)
    pooled_ref[0, 0] = jnp.max(z, axis=0)


def _run_fusion(fcat, wf, bf):
    bsz, n, cc = fcat.shape
    return pl.pallas_call(
        _fusion_kernel,
        grid=(bsz,),
        in_specs=[
            pl.BlockSpec((1, n, cc), lambda b_: (b_, 0, 0)),
            pl.BlockSpec((cc, _CF), lambda b_: (0, 0)),
            pl.BlockSpec((1, _CF), lambda b_: (0, 0)),
        ],
        out_specs=pl.BlockSpec((1, 1, _CF), lambda b_: (b_, 0, 0)),
        out_shape=jax.ShapeDtypeStruct((bsz, 1, _CF), _F32),
        compiler_params=pltpu.CompilerParams(
            dimension_semantics=("parallel",)),
    )(fcat, wf, bf)


def kernel(inputs, W_head, b_head, W0, b0, W1, b1, W2, b2, W_fuse, b_fuse):
    x = inputs[..., 0]
    xt = jnp.transpose(x, (0, 2, 1))
    pad = lambda a: jnp.pad(a, ((0, 0), (0, 0), (0, 128 - a.shape[-1])))
    padw = lambda w: jnp.pad(w, ((0, _C - w.shape[0]), (0, 0)))
    xt_pad = pad(xt)
    w1h, w2h = padw(W_head[:3]), padw(W_head[3:])

    zeros = jnp.zeros((_B, _N, _C), _F32)
    f0 = _stage(xt_pad, xt, zeros, w1h, w2h, b_head.reshape(1, -1),
                d=1, residual=False)
    f1 = _stage(pad(f0), f0, f0, W0[:_C], W0[_C:],
                b0.reshape(1, -1), d=1, residual=True)
    f2 = _stage(pad(f1), f1, f1, W1[:_C], W1[_C:],
                b1.reshape(1, -1), d=2, residual=True)
    f3 = _stage(pad(f2), f2, f2, W2[:_C], W2[_C:],
                b2.reshape(1, -1), d=3, residual=True)

    fcat = jnp.concatenate([f0, f1, f2, f3], axis=-1)
    pooled = _run_fusion(fcat, W_fuse, b_fuse.reshape(1, -1))

    fcatT = jnp.transpose(fcat, (0, 2, 1))
    fus = jnp.broadcast_to(pooled[:, 0, :, None], (_B, _CF, _N))
    out = jnp.concatenate([fus, fcatT], axis=1)
    return out[..., None]
